# Initial kernel scaffold; baseline (speedup 1.0000x reference)
import jax
import jax.numpy as jnp
from jax.experimental import pallas as pl


def kernel(z_e_x, emb_weight):
    raise NotImplementedError("write your pallas kernel here")


# speedup vs baseline: 1.9608x; 1.9608x over previous
import functools

import jax
import jax.numpy as jnp
from jax.experimental import pallas as pl

K = 8192
D = 256
TN = 256


def _vq_kernel(z_ref, e_ref, zn_ref, en_ref, out_ref):
    z = z_ref[...]
    e = e_ref[...]
    zn = zn_ref[...]
    en = en_ref[...]
    scores = jax.lax.dot_general(
        z, e, (((1,), (1,)), ((), ())), preferred_element_type=jnp.float32
    )
    dists = zn + en - 2.0 * scores
    m = jnp.min(dists, axis=1, keepdims=True)
    ks = jax.lax.broadcasted_iota(jnp.int32, dists.shape, 1)
    idx = jnp.min(jnp.where(dists == m, ks, dists.shape[1]), axis=1)
    out_ref[...] = idx.astype(jnp.int32)


def kernel(z_e_x, emb_weight):
    B, Dm, H, W = z_e_x.shape
    N = B * H * W
    z_r = jnp.transpose(z_e_x, (0, 2, 3, 1)).reshape(N, Dm)
    z_norm = (z_r ** 2).sum(axis=1, keepdims=True)
    e_norm = (emb_weight ** 2).sum(axis=1, keepdims=True).T

    latents = pl.pallas_call(
        _vq_kernel,
        grid=(N // TN,),
        in_specs=[
            pl.BlockSpec((TN, D), lambda i: (i, 0)),
            pl.BlockSpec((K, D), lambda i: (0, 0)),
            pl.BlockSpec((TN, 1), lambda i: (i, 0)),
            pl.BlockSpec((1, K), lambda i: (0, 0)),
        ],
        out_specs=pl.BlockSpec((TN,), lambda i: (i,)),
        out_shape=jax.ShapeDtypeStruct((N,), jnp.int32),
    )(z_r, emb_weight, z_norm, e_norm)

    return latents.reshape(B, H, W)


# speedup vs baseline: 2.1712x; 1.1073x over previous
import functools

import jax
import jax.numpy as jnp
from jax.experimental import pallas as pl

K = 8192
D = 256
TN = 512


def _vq_kernel(z_ref, e_ref, zn_ref, en_ref, out_ref):
    z = z_ref[...]
    e = e_ref[...]
    zn = zn_ref[...]
    en = en_ref[...]
    scores = jax.lax.dot_general(
        z, e, (((1,), (1,)), ((), ())), preferred_element_type=jnp.float32
    )
    dists = zn + en - 2.0 * scores
    m = jnp.min(dists, axis=1, keepdims=True)
    ks = jax.lax.broadcasted_iota(jnp.int32, dists.shape, 1)
    idx = jnp.min(jnp.where(dists == m, ks, dists.shape[1]), axis=1)
    out_ref[...] = idx.astype(jnp.int32)


def kernel(z_e_x, emb_weight):
    B, Dm, H, W = z_e_x.shape
    N = B * H * W
    z_r = jnp.transpose(z_e_x, (0, 2, 3, 1)).reshape(N, Dm)
    z_norm = (z_r ** 2).sum(axis=1, keepdims=True)
    e_norm = (emb_weight ** 2).sum(axis=1, keepdims=True).T

    latents = pl.pallas_call(
        _vq_kernel,
        grid=(N // TN,),
        in_specs=[
            pl.BlockSpec((TN, D), lambda i: (i, 0)),
            pl.BlockSpec((K, D), lambda i: (0, 0)),
            pl.BlockSpec((TN, 1), lambda i: (i, 0)),
            pl.BlockSpec((1, K), lambda i: (0, 0)),
        ],
        out_specs=pl.BlockSpec((TN,), lambda i: (i,)),
        out_shape=jax.ShapeDtypeStruct((N,), jnp.int32),
    )(z_r, emb_weight, z_norm, e_norm)

    return latents.reshape(B, H, W)


# speedup vs baseline: 2.2469x; 1.0349x over previous
import functools

import jax
import jax.numpy as jnp
from jax.experimental import pallas as pl

K = 8192
D = 256
TN = 1024


def _vq_kernel(z_ref, e_ref, zn_ref, en_ref, out_ref):
    z = z_ref[...]
    e = e_ref[...]
    zn = zn_ref[...]
    en = en_ref[...]
    scores = jax.lax.dot_general(
        z, e, (((1,), (1,)), ((), ())), preferred_element_type=jnp.float32
    )
    dists = zn + en - 2.0 * scores
    m = jnp.min(dists, axis=1, keepdims=True)
    ks = jax.lax.broadcasted_iota(jnp.int32, dists.shape, 1)
    idx = jnp.min(jnp.where(dists == m, ks, dists.shape[1]), axis=1)
    out_ref[...] = idx.astype(jnp.int32)


def kernel(z_e_x, emb_weight):
    B, Dm, H, W = z_e_x.shape
    N = B * H * W
    z_r = jnp.transpose(z_e_x, (0, 2, 3, 1)).reshape(N, Dm)
    z_norm = (z_r ** 2).sum(axis=1, keepdims=True)
    e_norm = (emb_weight ** 2).sum(axis=1, keepdims=True).T

    latents = pl.pallas_call(
        _vq_kernel,
        grid=(N // TN,),
        in_specs=[
            pl.BlockSpec((TN, D), lambda i: (i, 0)),
            pl.BlockSpec((K, D), lambda i: (0, 0)),
            pl.BlockSpec((TN, 1), lambda i: (i, 0)),
            pl.BlockSpec((1, K), lambda i: (0, 0)),
        ],
        out_specs=pl.BlockSpec((TN,), lambda i: (i,)),
        out_shape=jax.ShapeDtypeStruct((N,), jnp.int32),
    )(z_r, emb_weight, z_norm, e_norm)

    return latents.reshape(B, H, W)
